# initial kernel scaffold (unmeasured)
import jax
import jax.numpy as jnp
from jax import lax
from jax.experimental import pallas as pl
from jax.experimental.pallas import tpu as pltpu


def kernel(
    x,
):
    def body(*refs):
        pass

    out_shape = jax.ShapeDtypeStruct(..., jnp.float32)
    return pl.pallas_call(body, out_shape=out_shape)(...)


# baseline (device time: 407356 ns/iter reference)
import jax
import jax.numpy as jnp
from jax import lax
from jax.experimental import pallas as pl
from jax.experimental.pallas import tpu as pltpu


def kernel(x):
    xb = x[0].astype(jnp.bfloat16)
    m, n = xb.shape

    def body(x_ref, out_ref, xrecv_ref, sem_x, sem_y, copy_sem):
        my_x = lax.axis_index("x")
        my_y = lax.axis_index("y")
        other_x = 1 - my_x
        other_y = 1 - my_y

        barrier_sem = pltpu.get_barrier_semaphore()
        pl.semaphore_signal(
            barrier_sem, inc=1,
            device_id=(other_x, my_y), device_id_type=pl.DeviceIdType.MESH,
        )
        pl.semaphore_signal(
            barrier_sem, inc=1,
            device_id=(my_x, other_y), device_id_type=pl.DeviceIdType.MESH,
        )
        pl.semaphore_wait(barrier_sem, 2)

        rdma_x = pltpu.make_async_remote_copy(
            src_ref=x_ref,
            dst_ref=xrecv_ref,
            send_sem=sem_x.at[0],
            recv_sem=sem_x.at[1],
            device_id=(other_x, my_y),
            device_id_type=pl.DeviceIdType.MESH,
        )
        rdma_x.start()
        rdma_x.wait()

        x_ref[...] = x_ref[...] + xrecv_ref[...]
        copy = pltpu.make_async_copy(
            x_ref, out_ref.at[:, pl.ds(my_y * n, n)], copy_sem
        )
        copy.start()
        copy.wait()

        rdma_y = pltpu.make_async_remote_copy(
            src_ref=out_ref.at[:, pl.ds(my_y * n, n)],
            dst_ref=out_ref.at[:, pl.ds(my_y * n, n)],
            send_sem=sem_y.at[0],
            recv_sem=sem_y.at[1],
            device_id=(my_x, other_y),
            device_id_type=pl.DeviceIdType.MESH,
        )
        rdma_y.start()
        rdma_y.wait()

    return pl.pallas_call(
        body,
        out_shape=jax.ShapeDtypeStruct((m, 2 * n), jnp.bfloat16),
        in_specs=[pl.BlockSpec(memory_space=pltpu.VMEM)],
        out_specs=pl.BlockSpec(memory_space=pltpu.MemorySpace.HBM),
        scratch_shapes=[
            pltpu.VMEM((m, n), jnp.bfloat16),
            pltpu.SemaphoreType.DMA((2,)),
            pltpu.SemaphoreType.DMA((2,)),
            pltpu.SemaphoreType.DMA,
        ],
        compiler_params=pltpu.CompilerParams(collective_id=0),
    )(xb)


# device time: 242164 ns/iter; 1.6821x vs baseline; 1.6821x over previous
import jax
import jax.numpy as jnp
from jax import lax
from jax.experimental import pallas as pl
from jax.experimental.pallas import tpu as pltpu

NC = 8


def kernel(x):
    xb = x[0].astype(jnp.bfloat16)
    m, n = xb.shape
    r = m // NC

    def body(
        x_ref,
        out_ref,
        xrecv_ref,
        xsend_sems,
        xrecv_sems,
        ysend_sems,
        yrecv_sems,
        copy_sems,
    ):
        my_x = lax.axis_index("x")
        my_y = lax.axis_index("y")
        other_x = 1 - my_x
        other_y = 1 - my_y

        def x_rdma(c):
            rows = pl.ds(c * r, r)
            return pltpu.make_async_remote_copy(
                src_ref=x_ref.at[rows],
                dst_ref=xrecv_ref.at[rows],
                send_sem=xsend_sems.at[c],
                recv_sem=xrecv_sems.at[c],
                device_id=(other_x, my_y),
                device_id_type=pl.DeviceIdType.MESH,
            )

        def y_rdma(c):
            rows = pl.ds(c * r, r)
            return pltpu.make_async_remote_copy(
                src_ref=xrecv_ref.at[rows],
                dst_ref=out_ref.at[rows, pl.ds(my_y * n, n)],
                send_sem=ysend_sems.at[c],
                recv_sem=yrecv_sems.at[c],
                device_id=(my_x, other_y),
                device_id_type=pl.DeviceIdType.MESH,
            )

        def local_copy(c):
            rows = pl.ds(c * r, r)
            return pltpu.make_async_copy(
                xrecv_ref.at[rows],
                out_ref.at[rows, pl.ds(my_y * n, n)],
                copy_sems.at[c],
            )

        barrier_sem = pltpu.get_barrier_semaphore()
        pl.semaphore_signal(
            barrier_sem, inc=1,
            device_id=(other_x, my_y), device_id_type=pl.DeviceIdType.MESH,
        )
        pl.semaphore_signal(
            barrier_sem, inc=1,
            device_id=(my_x, other_y), device_id_type=pl.DeviceIdType.MESH,
        )
        pl.semaphore_wait(barrier_sem, 2)

        for c in range(NC):
            x_rdma(c).start()

        for c in range(NC):
            rows = pl.ds(c * r, r)
            x_rdma(c).wait_recv()
            xrecv_ref[rows, :] = xrecv_ref[rows, :] + x_ref[rows, :]
            y_rdma(c).start()
            local_copy(c).start()

        for c in range(NC):
            x_rdma(c).wait_send()
            yd = y_rdma(c)
            yd.wait_send()
            yd.wait_recv()
            local_copy(c).wait()

    return pl.pallas_call(
        body,
        out_shape=jax.ShapeDtypeStruct((m, 2 * n), jnp.bfloat16),
        in_specs=[pl.BlockSpec(memory_space=pltpu.VMEM)],
        out_specs=pl.BlockSpec(memory_space=pltpu.MemorySpace.HBM),
        scratch_shapes=[
            pltpu.VMEM((m, n), jnp.bfloat16),
            pltpu.SemaphoreType.DMA((NC,)),
            pltpu.SemaphoreType.DMA((NC,)),
            pltpu.SemaphoreType.DMA((NC,)),
            pltpu.SemaphoreType.DMA((NC,)),
            pltpu.SemaphoreType.DMA((NC,)),
        ],
        compiler_params=pltpu.CompilerParams(collective_id=0),
    )(xb)


# device time: 221655 ns/iter; 1.8378x vs baseline; 1.0925x over previous
import jax
import jax.numpy as jnp
from jax import lax
from jax.experimental import pallas as pl
from jax.experimental.pallas import tpu as pltpu

NC = 16


def kernel(x):
    _, m, n = x.shape
    r = m // NC

    def body(
        x_ref,
        out_ref,
        xb_ref,
        xrecv_ref,
        stage_ref,
        load_sems,
        xsend_sems,
        xrecv_sems,
        ysend_sems,
        yrecv_sems,
        copy_sems,
    ):
        my_x = lax.axis_index("x")
        my_y = lax.axis_index("y")
        other_x = 1 - my_x
        other_y = 1 - my_y

        def load(c):
            return pltpu.make_async_copy(
                x_ref.at[0, pl.ds(c * r, r)],
                stage_ref.at[c % 2],
                load_sems.at[c % 2],
            )

        def x_rdma(c):
            rows = pl.ds(c * r, r)
            return pltpu.make_async_remote_copy(
                src_ref=xb_ref.at[rows],
                dst_ref=xrecv_ref.at[rows],
                send_sem=xsend_sems.at[c],
                recv_sem=xrecv_sems.at[c],
                device_id=(other_x, my_y),
                device_id_type=pl.DeviceIdType.MESH,
            )

        def y_rdma(c):
            rows = pl.ds(c * r, r)
            return pltpu.make_async_remote_copy(
                src_ref=xrecv_ref.at[rows],
                dst_ref=out_ref.at[rows, pl.ds(my_y * n, n)],
                send_sem=ysend_sems.at[c],
                recv_sem=yrecv_sems.at[c],
                device_id=(my_x, other_y),
                device_id_type=pl.DeviceIdType.MESH,
            )

        def local_copy(c):
            rows = pl.ds(c * r, r)
            return pltpu.make_async_copy(
                xrecv_ref.at[rows],
                out_ref.at[rows, pl.ds(my_y * n, n)],
                copy_sems.at[c],
            )

        barrier_sem = pltpu.get_barrier_semaphore()
        pl.semaphore_signal(
            barrier_sem, inc=1,
            device_id=(other_x, my_y), device_id_type=pl.DeviceIdType.MESH,
        )
        pl.semaphore_signal(
            barrier_sem, inc=1,
            device_id=(my_x, other_y), device_id_type=pl.DeviceIdType.MESH,
        )
        pl.semaphore_wait(barrier_sem, 2)

        load(0).start()
        load(1).start()
        for c in range(NC):
            rows = pl.ds(c * r, r)
            load(c).wait()
            xb_ref[rows, :] = stage_ref[c % 2].astype(jnp.bfloat16)
            x_rdma(c).start()
            if c + 2 < NC:
                load(c + 2).start()

        for c in range(NC):
            rows = pl.ds(c * r, r)
            x_rdma(c).wait_recv()
            xrecv_ref[rows, :] = xrecv_ref[rows, :] + xb_ref[rows, :]
            y_rdma(c).start()
            local_copy(c).start()

        for c in range(NC):
            x_rdma(c).wait_send()
            yd = y_rdma(c)
            yd.wait_send()
            yd.wait_recv()
            local_copy(c).wait()

    return pl.pallas_call(
        body,
        out_shape=jax.ShapeDtypeStruct((m, 2 * n), jnp.bfloat16),
        in_specs=[pl.BlockSpec(memory_space=pltpu.MemorySpace.HBM)],
        out_specs=pl.BlockSpec(memory_space=pltpu.MemorySpace.HBM),
        scratch_shapes=[
            pltpu.VMEM((m, n), jnp.bfloat16),
            pltpu.VMEM((m, n), jnp.bfloat16),
            pltpu.VMEM((2, m // NC, n), jnp.float32),
            pltpu.SemaphoreType.DMA((2,)),
            pltpu.SemaphoreType.DMA((NC,)),
            pltpu.SemaphoreType.DMA((NC,)),
            pltpu.SemaphoreType.DMA((NC,)),
            pltpu.SemaphoreType.DMA((NC,)),
            pltpu.SemaphoreType.DMA((NC,)),
        ],
        compiler_params=pltpu.CompilerParams(
            collective_id=0, vmem_limit_bytes=48 * 1024 * 1024
        ),
    )(x)
